# baseline (speedup 1.0000x reference)
import jax
import jax.numpy as jnp
from jax.experimental import pallas as pl
from jax.experimental.pallas import tpu as pltpu

B, T, D, E = 4, 4096, 4096, 64
ROWS = 16384
BLK = 1024


def _router_kernel(x_ref, sig_ref, idx_ref, probs_ref, sgn_ref):
    @pl.when(pl.program_id(0) == 0)
    def _():
        sgn_ref[...] = jnp.sign(sig_ref[...]).astype(jnp.bfloat16)

    xb = x_ref[...].astype(jnp.bfloat16)
    scores = jax.lax.dot_general(
        xb, sgn_ref[...], (((1,), (1,)), ((), ())),
        preferred_element_type=jnp.float32)

    m = jnp.max(scores, axis=1, keepdims=True)
    col = jax.lax.broadcasted_iota(jnp.int32, scores.shape, 1)
    idx_ref[...] = jnp.min(jnp.where(scores == m, col, E), axis=1)

    e = jnp.exp(scores - m)
    probs_ref[...] = (e / jnp.sum(e, axis=1, keepdims=True)).reshape(
        probs_ref.shape)


def kernel(x, signatures):
    x2 = x.reshape(ROWS, D)

    grid = (ROWS // BLK,)
    idx, probs = pl.pallas_call(
        _router_kernel,
        grid=grid,
        in_specs=[
            pl.BlockSpec((BLK, D), lambda i: (i, 0)),
            pl.BlockSpec((E, D), lambda i: (0, 0)),
        ],
        out_specs=[
            pl.BlockSpec((BLK,), lambda i: (i,)),
            pl.BlockSpec((1, BLK, E), lambda i: (i // (T // BLK), i % (T // BLK), 0)),
        ],
        out_shape=[
            jax.ShapeDtypeStruct((ROWS,), jnp.int32),
            jax.ShapeDtypeStruct((B, T, E), jnp.float32),
        ],
        scratch_shapes=[pltpu.VMEM((E, D), jnp.bfloat16)],
    )(x2, signatures)

    return idx.reshape(B, T), probs
